# baseline (speedup 1.0000x reference)
import functools

import jax
import jax.numpy as jnp
from jax import lax
from jax.experimental import pallas as pl
from jax.experimental.pallas import tpu as pltpu
from jax.experimental.pallas import tpu_sc as plsc

N = 10000
E = 160000
BATCH = 64
H = 128
LAT = 128
F_L = 10
F_P = 31
NLAYERS = 4


_TILES = 16
_EPT = E // _TILES
_CH = 40
_NPASS = 10
_WIN = 25
_NBUF = 5
_GRP = _WIN // _NBUF
_RCH = _NBUF * _CH
_NRCH = N // _RCH
_RTRIP = -(-_NRCH // _TILES)


def _sc_seg_body(h_l, h_p, lsrc, ldst, psrc, pdst, s_l, s_p,
                 sidx, didx, rows, acc, *sems):
    gsem = sems[:_NBUF]
    ssem = sems[_NBUF:2 * _NBUF]
    isem = sems[2 * _NBUF]
    c = lax.axis_index("c")
    s = lax.axis_index("s")

    def rbuf(t):
        return rows.at[pl.ds(t * _CH, _CH), :]

    def zfill(i, _):
        rows[i // 8, pl.ds((i % 8) * 16, 16)] = jnp.zeros((16,), jnp.float32)
        return 0
    lax.fori_loop(0, _RCH * 8, zfill, 0)

    def zchunk(j, _):
        ck = s + j * _TILES

        @pl.when(ck < _NRCH)
        def _():
            pltpu.sync_copy(rows, acc.at[pl.ds(ck * _RCH, _RCH), :])
        return 0
    lax.fori_loop(0, _RTRIP, zchunk, 0)
    plsc.subcore_barrier()

    def process(h_hbm, src_hbm, dst_hbm, out_hbm):
        def fire_gathers(sl, g):
            for t in range(_NBUF):
                pltpu.async_copy(h_hbm.at[sidx.at[sl, g * _NBUF + t]],
                                 rbuf(t), gsem[t])

        def wait_scatters():
            for t in range(_NBUF):
                pltpu.make_async_copy(rbuf(t), acc.at[didx.at[0, 0]],
                                      ssem[t]).wait()

        def finish_group(sl, g):
            for t in range(_NBUF):
                j = g * _NBUF + t
                pltpu.make_async_copy(h_hbm.at[sidx.at[sl, j]], rbuf(t),
                                      gsem[t]).wait()
                pltpu.async_copy(rbuf(t), acc.at[didx.at[sl, j]], ssem[t],
                                 add=True)

        pltpu.sync_copy(src_hbm.at[s, 0], sidx.at[0])
        pltpu.sync_copy(dst_hbm.at[s, 0], didx.at[0])
        for p in range(_NPASS):
            sl = p % 2
            if p == 0:
                fire_gathers(0, 0)
            else:
                pltpu.make_async_copy(src_hbm.at[s, p], sidx.at[sl],
                                      isem).wait()
                pltpu.make_async_copy(dst_hbm.at[s, p], didx.at[sl],
                                      isem).wait()
                wait_scatters()
                fire_gathers(sl, 0)
            finish_group(sl, 0)
            if p + 1 < _NPASS:
                nsl = 1 - sl
                pltpu.async_copy(src_hbm.at[s, p + 1], sidx.at[nsl], isem)
                pltpu.async_copy(dst_hbm.at[s, p + 1], didx.at[nsl], isem)

            def group(g, _):
                wait_scatters()
                fire_gathers(sl, g)
                finish_group(sl, g)
                return 0
            lax.fori_loop(1, _GRP, group, 0)
        wait_scatters()
        plsc.subcore_barrier()

        def wout(j, _):
            ck = s + j * _TILES

            @pl.when(ck < _NRCH)
            def _():
                r = ck * _RCH
                pltpu.sync_copy(acc.at[pl.ds(r, _RCH), :],
                                out_hbm.at[pl.ds(r, _RCH), :])
            return 0
        lax.fori_loop(0, _RTRIP, wout, 0)

    @pl.when(c == 0)
    def _():
        process(h_l, lsrc, ldst, s_l)

    @pl.when(c == 1)
    def _():
        process(h_p, psrc, pdst, s_p)


@functools.lru_cache(maxsize=1)
def _get_sc_seg():
    return pl.kernel(
        _sc_seg_body,
        out_type=(jax.ShapeDtypeStruct((N, H), jnp.float32),
                  jax.ShapeDtypeStruct((N, H), jnp.float32)),
        mesh=plsc.VectorSubcoreMesh(core_axis_name="c", subcore_axis_name="s",
                                    num_cores=2, num_subcores=_TILES),
        scratch_types=[
            pltpu.VMEM((2, _WIN, _CH), jnp.int32),
            pltpu.VMEM((2, _WIN, _CH), jnp.int32),
            pltpu.VMEM((_NBUF * _CH, H), jnp.float32),
            pltpu.VMEM_SHARED((N, H), jnp.float32),
        ] + [pltpu.SemaphoreType.DMA] * (2 * _NBUF + 1),
    )


def _sc_seg(*args):
    return _get_sc_seg()(*args)


_GRID = 10
_RB = N // _GRID


def _silu(x):
    return x * jax.nn.sigmoid(x)


def _dot(a, b):
    return jnp.dot(a, b, preferred_element_type=jnp.float32)


def _log_softmax(x):
    m = jnp.max(x, axis=-1, keepdims=True)
    s = jnp.sum(jnp.exp(x - m), axis=-1, keepdims=True)
    return x - m - jnp.log(s)


def _embed_body(lx, px, Wle, ble, Wpe, bpe, hl, hp):
    hl[...] = _dot(lx[...], Wle[...]) + ble[...]
    hp[...] = _dot(px[...], Wpe[...]) + bpe[...]


def _update_body(hl, sl, Wl, hp, sp, Wp, hlo, hpo):
    hlo[...] = _silu(hl[...] + _dot(sl[...], Wl[...]))
    hpo[...] = _silu(hp[...] + _dot(sp[...], Wp[...]))


def _heads(i, i0, hv, batch, to_v, et, te_all,
           tW0, tb0, tW1, tb1, tW2, tb2,
           dW0, db0, dW1, db1, dW2, db2,
           t_out, loss_out, d_out):
    onehot = (batch[:, None]
              == lax.broadcasted_iota(jnp.int32, (_RB, BATCH), 1)
              ).astype(jnp.float32)

    d = jnp.zeros((_RB, F_L), jnp.float32)
    for t in range(F_L):
        x = hv * et[t:t + 1, :]
        y = _silu(_dot(x, tW0[...]) + tb0[...])
        z = _silu(_dot(y, tW1[...]) + tb1[...])
        sel = (lax.broadcasted_iota(jnp.int32, (1, F_L), 1) == t
               ).astype(jnp.float32)
        d = d + _dot(z, tW2[...] * sel)
    d = d + tb2[...]
    nt = _log_softmax(d)
    agg_blk = lax.dot_general(onehot, nt, (((0,), (0,)), ((), ())),
                              preferred_element_type=jnp.float32)

    @pl.when(i == i0)
    def _():
        t_out[...] = agg_blk

    @pl.when(i > i0)
    def _():
        t_out[...] = t_out[...] + agg_blk

    @pl.when(i == i0 + _GRID - 1)
    def _():
        res = _log_softmax(t_out[...])
        t_out[...] = res
        safe = jnp.where(to_v > 0, to_v, 1.0)
        loss_out[...] = jnp.where(to_v > 0, to_v * (jnp.log(safe) - res), 0.0)

    te = _dot(onehot, te_all)
    x = hv * te
    y = _silu(_dot(x, dW0[...]) + db0[...])
    z = _silu(_dot(y, dW1[...]) + db1[...])
    d_out[...] = _log_softmax(_dot(z, dW2[...]) + db2[...])


def _final_body(hl, sl, Wl, hp, sp, Wp, lb3, pb3, to, Wle, ble,
                *rest):
    tlws = rest[:12]
    tpws = rest[12:24]
    Wm, bm, Wv, bv, eps, Wlat, blat = rest[24:31]
    (tll, tll_loss, dll, tlp, tlp_loss, dlp, vae) = rest[31:38]
    racc, etbuf, tebuf, lbuf = rest[38:42]
    i = pl.program_id(0)

    @pl.when(i == 0)
    def _():
        etbuf[...] = Wle[...] + ble[...]
        tebuf[...] = _dot(to[...], Wle[...]) + ble[...]

    @pl.when(i < _GRID)
    def _():
        hlv = _silu(hl[...] + _dot(sl[...], Wl[...]))
        hpv = _silu(hp[...] + _dot(sp[...], Wp[...]))
        part = (jnp.sum(hlv, axis=0, keepdims=True)
                + jnp.sum(hpv, axis=0, keepdims=True))

        @pl.when(i == 0)
        def _():
            racc[...] = part

        @pl.when(i > 0)
        def _():
            racc[...] = racc[...] + part

        _heads(i, 0, hlv, lb3[0, 0, :], to[...], etbuf[...], tebuf[...],
               *tlws, tll, tll_loss, dll)

    @pl.when(i == _GRID - 1)
    def _():
        readout = racc[...] * (1.0 / (2 * N))
        mean = _dot(readout, Wm[...]) + bm[...]
        logvar = _dot(readout, Wv[...]) + bv[...]
        latent = eps[...] * jnp.exp(0.5 * logvar) + mean
        vae[...] = -0.5 * jnp.sum(
            1.0 + logvar - mean * mean - jnp.exp(logvar),
            axis=-1, keepdims=True)
        lbuf[...] = _dot(latent, Wlat[H:, :]) + blat[...]

    @pl.when(i >= _GRID)
    def _():
        hpv = _silu(hp[...] + _dot(sp[...], Wp[...]))
        hv = _dot(hpv, Wlat[:H, :]) + lbuf[...]
        _heads(i, _GRID, hv, pb3[0, 0, :], to[...], etbuf[...], tebuf[...],
               *tpws, tlp, tlp_loss, dlp)


def _row_spec(cols):
    return pl.BlockSpec((_RB, cols), lambda i: (i, 0))


def _whole(shape):
    return pl.BlockSpec(shape, lambda i: tuple(0 for _ in shape))


def _embed_call(lx, px, Wle, ble, Wpe, bpe):
    return pl.pallas_call(
        _embed_body,
        grid=(_GRID,),
        in_specs=[_row_spec(F_L), _row_spec(F_P),
                  _whole((F_L, H)), _whole((1, H)),
                  _whole((F_P, H)), _whole((1, H))],
        out_specs=[_row_spec(H), _row_spec(H)],
        out_shape=[jax.ShapeDtypeStruct((N, H), jnp.float32)] * 2,
    )(lx, px, Wle, ble, Wpe, bpe)


def _update_call(hl, sl, Wl, hp, sp, Wp):
    return pl.pallas_call(
        _update_body,
        grid=(_GRID,),
        in_specs=[_row_spec(H), _row_spec(H), _whole((H, H))] * 2,
        out_specs=[_row_spec(H), _row_spec(H)],
        out_shape=[jax.ShapeDtypeStruct((N, H), jnp.float32)] * 2,
    )(hl, sl, Wl, hp, sp, Wp)


_T_DIMS = (128, 85, 43, 1)
_D_DIMS = (128, 93, 59, 25)


def _head_specs():
    specs = []
    for dims in (_T_DIMS, _D_DIMS):
        for j in range(3):
            specs.append(_whole((dims[j], dims[j + 1])))
            specs.append(_whole((1, dims[j + 1])))
    return specs


def _final_call(hl, sl, Wl, hp, sp, Wp, lb3, pb3, to, Wle, ble,
                tlws, tpws, Wm, bm, Wv, bv, eps, Wlat, blat):
    lig = lambda i: (jnp.minimum(i, _GRID - 1), 0)
    both = lambda i: (lax.rem(i, _GRID), 0)
    return pl.pallas_call(
        _final_body,
        grid=(2 * _GRID,),
        in_specs=[pl.BlockSpec((_RB, H), lig), pl.BlockSpec((_RB, H), lig),
                  _whole((H, H)),
                  pl.BlockSpec((_RB, H), both), pl.BlockSpec((_RB, H), both),
                  _whole((H, H)),
                  pl.BlockSpec((1, 1, _RB),
                               lambda i: (jnp.minimum(i, _GRID - 1), 0, 0)),
                  pl.BlockSpec((1, 1, _RB),
                               lambda i: (lax.rem(i, _GRID), 0, 0)),
                  _whole((BATCH, F_L)),
                  _whole((F_L, H)), _whole((1, H))]
                 + _head_specs() + _head_specs()
                 + [_whole((H, LAT)), _whole((1, LAT)),
                    _whole((H, LAT)), _whole((1, LAT)),
                    _whole((1, LAT)),
                    _whole((H + LAT, H)), _whole((1, H))],
        out_specs=[_whole((BATCH, F_L)), _whole((BATCH, F_L)),
                   pl.BlockSpec((_RB, _D_DIMS[3]), lig),
                   _whole((BATCH, F_L)), _whole((BATCH, F_L)),
                   pl.BlockSpec((_RB, _D_DIMS[3]),
                                lambda i: (jnp.maximum(i, _GRID) - _GRID, 0)),
                   _whole((1, 1))],
        out_shape=[jax.ShapeDtypeStruct((BATCH, F_L), jnp.float32),
                   jax.ShapeDtypeStruct((BATCH, F_L), jnp.float32),
                   jax.ShapeDtypeStruct((N, _D_DIMS[3]), jnp.float32),
                   jax.ShapeDtypeStruct((BATCH, F_L), jnp.float32),
                   jax.ShapeDtypeStruct((BATCH, F_L), jnp.float32),
                   jax.ShapeDtypeStruct((N, _D_DIMS[3]), jnp.float32),
                   jax.ShapeDtypeStruct((1, 1), jnp.float32)],
        scratch_shapes=[pltpu.VMEM((1, LAT), jnp.float32),
                        pltpu.VMEM((F_L, H), jnp.float32),
                        pltpu.VMEM((BATCH, H), jnp.float32),
                        pltpu.VMEM((1, H), jnp.float32)],
    )(hl, sl, Wl, hp, sp, Wp, lb3, pb3, to, Wle, ble, *tlws, *tpws,
      Wm, bm, Wv, bv, eps, Wlat, blat)


def kernel(ligand_x, pocket_x, type_output, eps, ligand_batch, pocket_batch,
           ll_edge_index, pp_edge_index, params):
    p = params
    r2 = lambda b: b.reshape(1, -1)

    def idx3(row):
        return row.astype(jnp.int32).reshape(_TILES, _NPASS, _WIN, _CH)

    lsrc = idx3(ll_edge_index[0])
    ldst = idx3(ll_edge_index[1])
    psrc = idx3(pp_edge_index[0])
    pdst = idx3(pp_edge_index[1])

    h_l, h_p = _embed_call(ligand_x, pocket_x,
                           p["Wle"], r2(p["ble"]), p["Wpe"], r2(p["bpe"]))

    for i in range(NLAYERS - 1):
        s_l, s_p = _sc_seg(h_l, h_p, lsrc, ldst, psrc, pdst)
        h_l, h_p = _update_call(h_l, s_l, p["Wg_l"][i], h_p, s_p, p["Wg_p"][i])
    s_l, s_p = _sc_seg(h_l, h_p, lsrc, ldst, psrc, pdst)

    lb3 = ligand_batch.astype(jnp.int32).reshape(_GRID, 1, _RB)
    pb3 = pocket_batch.astype(jnp.int32).reshape(_GRID, 1, _RB)

    def head_ws(pre):
        return tuple(x for j in range(3)
                     for x in (p[pre + "_W%d" % j], r2(p[pre + "_b%d" % j])))

    tll, tll_loss, dll, tlp, tlp_loss, dlp, vae = _final_call(
        h_l, s_l, p["Wg_l"][NLAYERS - 1], h_p, s_p, p["Wg_p"][NLAYERS - 1],
        lb3, pb3, type_output, p["Wle"], r2(p["ble"]),
        head_ws("ntll") + head_ws("ndll"), head_ws("ntlp") + head_ws("ndlp"),
        p["Wm"], r2(p["bm"]), p["Wv"], r2(p["bv"]), eps,
        p["Wlat"], r2(p["blat"]))
    vae_loss = vae.reshape(1)

    return (tll, tlp, tll_loss, tlp_loss, dll, dlp, vae_loss)


# speedup vs baseline: 1.1381x; 1.1381x over previous
import functools

import jax
import jax.numpy as jnp
from jax import lax
from jax.experimental import pallas as pl
from jax.experimental.pallas import tpu as pltpu
from jax.experimental.pallas import tpu_sc as plsc

N = 10000
E = 160000
BATCH = 64
H = 128
LAT = 128
F_L = 10
F_P = 31
NLAYERS = 4


_TILES = 16
_EPT = E // _TILES
_CH = 40
_NPASS = 5
_WIN = 50
_NBUF = 5
_GRP = _WIN // _NBUF
_RCH = _NBUF * _CH
_NRCH = N // _RCH
_RTRIP = -(-_NRCH // _TILES)


def _sc_seg_body(h_l, h_p, lsrc, ldst, psrc, pdst, s_l, s_p,
                 sidx, didx, rows, acc, *sems):
    gsem = sems[:_NBUF]
    ssem = sems[_NBUF:]
    c = lax.axis_index("c")
    s = lax.axis_index("s")

    def rbuf(t):
        return rows.at[pl.ds(t * _CH, _CH), :]

    def zfill(i, _):
        rows[i // 8, pl.ds((i % 8) * 16, 16)] = jnp.zeros((16,), jnp.float32)
        return 0
    lax.fori_loop(0, _RCH * 8, zfill, 0)

    def zchunk(j, _):
        ck = s + j * _TILES

        @pl.when(ck < _NRCH)
        def _():
            pltpu.sync_copy(rows, acc.at[pl.ds(ck * _RCH, _RCH), :])
        return 0
    lax.fori_loop(0, _RTRIP, zchunk, 0)
    plsc.subcore_barrier()

    def process(h_hbm, src_hbm, dst_hbm, out_hbm):
        for p in range(_NPASS):
            if p > 0:
                for t in range(_NBUF):
                    pltpu.make_async_copy(rbuf(t), acc.at[didx.at[0]],
                                          ssem[t]).wait()
            pltpu.sync_copy(src_hbm.at[s, p], sidx)
            pltpu.sync_copy(dst_hbm.at[s, p], didx)

            def group(g, _):
                for t in range(_NBUF):
                    j = g * _NBUF + t

                    @pl.when(g >= 1)
                    def _():
                        pltpu.make_async_copy(
                            rbuf(t), acc.at[didx.at[j - _NBUF]],
                            ssem[t]).wait()
                    pltpu.async_copy(h_hbm.at[sidx.at[j]], rbuf(t),
                                     gsem[t])
                for t in range(_NBUF):
                    j = g * _NBUF + t
                    pltpu.make_async_copy(h_hbm.at[sidx.at[j]], rbuf(t),
                                          gsem[t]).wait()
                    pltpu.async_copy(rbuf(t), acc.at[didx.at[j]], ssem[t],
                                     add=True)
                return 0
            lax.fori_loop(0, _GRP, group, 0)
        for t in range(_NBUF):
            pltpu.make_async_copy(rbuf(t), acc.at[didx.at[0]],
                                  ssem[t]).wait()
        plsc.subcore_barrier()

        def wout(j, _):
            ck = s + j * _TILES

            @pl.when(ck < _NRCH)
            def _():
                r = ck * _RCH
                pltpu.sync_copy(acc.at[pl.ds(r, _RCH), :],
                                out_hbm.at[pl.ds(r, _RCH), :])
            return 0
        lax.fori_loop(0, _RTRIP, wout, 0)

    @pl.when(c == 0)
    def _():
        process(h_l, lsrc, ldst, s_l)

    @pl.when(c == 1)
    def _():
        process(h_p, psrc, pdst, s_p)


@functools.lru_cache(maxsize=1)
def _get_sc_seg():
    return pl.kernel(
        _sc_seg_body,
        out_type=(jax.ShapeDtypeStruct((N, H), jnp.float32),
                  jax.ShapeDtypeStruct((N, H), jnp.float32)),
        mesh=plsc.VectorSubcoreMesh(core_axis_name="c", subcore_axis_name="s",
                                    num_cores=2, num_subcores=_TILES),
        scratch_types=[
            pltpu.VMEM((_WIN, _CH), jnp.int32),
            pltpu.VMEM((_WIN, _CH), jnp.int32),
            pltpu.VMEM((_NBUF * _CH, H), jnp.float32),
            pltpu.VMEM_SHARED((N, H), jnp.float32),
        ] + [pltpu.SemaphoreType.DMA] * (2 * _NBUF),
    )


def _sc_seg(*args):
    return _get_sc_seg()(*args)


_GRID = 10
_RB = N // _GRID


def _silu(x):
    return x * jax.nn.sigmoid(x)


def _dot(a, b):
    return jnp.dot(a, b, preferred_element_type=jnp.float32)


def _log_softmax(x):
    m = jnp.max(x, axis=-1, keepdims=True)
    s = jnp.sum(jnp.exp(x - m), axis=-1, keepdims=True)
    return x - m - jnp.log(s)


def _embed_body(lx, px, Wle, ble, Wpe, bpe, hl, hp):
    hl[...] = _dot(lx[...], Wle[...]) + ble[...]
    hp[...] = _dot(px[...], Wpe[...]) + bpe[...]


def _update_body(hl, sl, Wl, hp, sp, Wp, hlo, hpo):
    hlo[...] = _silu(hl[...] + _dot(sl[...], Wl[...]))
    hpo[...] = _silu(hp[...] + _dot(sp[...], Wp[...]))


def _heads(i, i0, hv, batch, to_v, et, te_all,
           tW0, tb0, tW1, tb1, tW2, tb2,
           dW0, db0, dW1, db1, dW2, db2,
           t_out, loss_out, d_out):
    onehot = (batch[:, None]
              == lax.broadcasted_iota(jnp.int32, (_RB, BATCH), 1)
              ).astype(jnp.float32)

    d = jnp.zeros((_RB, F_L), jnp.float32)
    for t in range(F_L):
        x = hv * et[t:t + 1, :]
        y = _silu(_dot(x, tW0[...]) + tb0[...])
        z = _silu(_dot(y, tW1[...]) + tb1[...])
        sel = (lax.broadcasted_iota(jnp.int32, (1, F_L), 1) == t
               ).astype(jnp.float32)
        d = d + _dot(z, tW2[...] * sel)
    d = d + tb2[...]
    nt = _log_softmax(d)
    agg_blk = lax.dot_general(onehot, nt, (((0,), (0,)), ((), ())),
                              preferred_element_type=jnp.float32)

    @pl.when(i == i0)
    def _():
        t_out[...] = agg_blk

    @pl.when(i > i0)
    def _():
        t_out[...] = t_out[...] + agg_blk

    @pl.when(i == i0 + _GRID - 1)
    def _():
        res = _log_softmax(t_out[...])
        t_out[...] = res
        safe = jnp.where(to_v > 0, to_v, 1.0)
        loss_out[...] = jnp.where(to_v > 0, to_v * (jnp.log(safe) - res), 0.0)

    te = _dot(onehot, te_all)
    x = hv * te
    y = _silu(_dot(x, dW0[...]) + db0[...])
    z = _silu(_dot(y, dW1[...]) + db1[...])
    d_out[...] = _log_softmax(_dot(z, dW2[...]) + db2[...])


def _final_body(hl, sl, Wl, hp, sp, Wp, lb3, pb3, to, Wle, ble,
                *rest):
    tlws = rest[:12]
    tpws = rest[12:24]
    Wm, bm, Wv, bv, eps, Wlat, blat = rest[24:31]
    (tll, tll_loss, dll, tlp, tlp_loss, dlp, vae) = rest[31:38]
    racc, etbuf, tebuf, lbuf = rest[38:42]
    i = pl.program_id(0)

    @pl.when(i == 0)
    def _():
        etbuf[...] = Wle[...] + ble[...]
        tebuf[...] = _dot(to[...], Wle[...]) + ble[...]

    @pl.when(i < _GRID)
    def _():
        hlv = _silu(hl[...] + _dot(sl[...], Wl[...]))
        hpv = _silu(hp[...] + _dot(sp[...], Wp[...]))
        part = (jnp.sum(hlv, axis=0, keepdims=True)
                + jnp.sum(hpv, axis=0, keepdims=True))

        @pl.when(i == 0)
        def _():
            racc[...] = part

        @pl.when(i > 0)
        def _():
            racc[...] = racc[...] + part

        _heads(i, 0, hlv, lb3[0, 0, :], to[...], etbuf[...], tebuf[...],
               *tlws, tll, tll_loss, dll)

    @pl.when(i == _GRID - 1)
    def _():
        readout = racc[...] * (1.0 / (2 * N))
        mean = _dot(readout, Wm[...]) + bm[...]
        logvar = _dot(readout, Wv[...]) + bv[...]
        latent = eps[...] * jnp.exp(0.5 * logvar) + mean
        vae[...] = -0.5 * jnp.sum(
            1.0 + logvar - mean * mean - jnp.exp(logvar),
            axis=-1, keepdims=True)
        lbuf[...] = _dot(latent, Wlat[H:, :]) + blat[...]

    @pl.when(i >= _GRID)
    def _():
        hpv = _silu(hp[...] + _dot(sp[...], Wp[...]))
        hv = _dot(hpv, Wlat[:H, :]) + lbuf[...]
        _heads(i, _GRID, hv, pb3[0, 0, :], to[...], etbuf[...], tebuf[...],
               *tpws, tlp, tlp_loss, dlp)


def _row_spec(cols):
    return pl.BlockSpec((_RB, cols), lambda i: (i, 0))


def _whole(shape):
    return pl.BlockSpec(shape, lambda i: tuple(0 for _ in shape))


def _embed_call(lx, px, Wle, ble, Wpe, bpe):
    return pl.pallas_call(
        _embed_body,
        grid=(_GRID,),
        in_specs=[_row_spec(F_L), _row_spec(F_P),
                  _whole((F_L, H)), _whole((1, H)),
                  _whole((F_P, H)), _whole((1, H))],
        out_specs=[_row_spec(H), _row_spec(H)],
        out_shape=[jax.ShapeDtypeStruct((N, H), jnp.float32)] * 2,
    )(lx, px, Wle, ble, Wpe, bpe)


def _update_call(hl, sl, Wl, hp, sp, Wp):
    return pl.pallas_call(
        _update_body,
        grid=(_GRID,),
        in_specs=[_row_spec(H), _row_spec(H), _whole((H, H))] * 2,
        out_specs=[_row_spec(H), _row_spec(H)],
        out_shape=[jax.ShapeDtypeStruct((N, H), jnp.float32)] * 2,
    )(hl, sl, Wl, hp, sp, Wp)


_T_DIMS = (128, 85, 43, 1)
_D_DIMS = (128, 93, 59, 25)


def _head_specs():
    specs = []
    for dims in (_T_DIMS, _D_DIMS):
        for j in range(3):
            specs.append(_whole((dims[j], dims[j + 1])))
            specs.append(_whole((1, dims[j + 1])))
    return specs


def _final_call(hl, sl, Wl, hp, sp, Wp, lb3, pb3, to, Wle, ble,
                tlws, tpws, Wm, bm, Wv, bv, eps, Wlat, blat):
    lig = lambda i: (jnp.minimum(i, _GRID - 1), 0)
    both = lambda i: (lax.rem(i, _GRID), 0)
    return pl.pallas_call(
        _final_body,
        grid=(2 * _GRID,),
        in_specs=[pl.BlockSpec((_RB, H), lig), pl.BlockSpec((_RB, H), lig),
                  _whole((H, H)),
                  pl.BlockSpec((_RB, H), both), pl.BlockSpec((_RB, H), both),
                  _whole((H, H)),
                  pl.BlockSpec((1, 1, _RB),
                               lambda i: (jnp.minimum(i, _GRID - 1), 0, 0)),
                  pl.BlockSpec((1, 1, _RB),
                               lambda i: (lax.rem(i, _GRID), 0, 0)),
                  _whole((BATCH, F_L)),
                  _whole((F_L, H)), _whole((1, H))]
                 + _head_specs() + _head_specs()
                 + [_whole((H, LAT)), _whole((1, LAT)),
                    _whole((H, LAT)), _whole((1, LAT)),
                    _whole((1, LAT)),
                    _whole((H + LAT, H)), _whole((1, H))],
        out_specs=[_whole((BATCH, F_L)), _whole((BATCH, F_L)),
                   pl.BlockSpec((_RB, _D_DIMS[3]), lig),
                   _whole((BATCH, F_L)), _whole((BATCH, F_L)),
                   pl.BlockSpec((_RB, _D_DIMS[3]),
                                lambda i: (jnp.maximum(i, _GRID) - _GRID, 0)),
                   _whole((1, 1))],
        out_shape=[jax.ShapeDtypeStruct((BATCH, F_L), jnp.float32),
                   jax.ShapeDtypeStruct((BATCH, F_L), jnp.float32),
                   jax.ShapeDtypeStruct((N, _D_DIMS[3]), jnp.float32),
                   jax.ShapeDtypeStruct((BATCH, F_L), jnp.float32),
                   jax.ShapeDtypeStruct((BATCH, F_L), jnp.float32),
                   jax.ShapeDtypeStruct((N, _D_DIMS[3]), jnp.float32),
                   jax.ShapeDtypeStruct((1, 1), jnp.float32)],
        scratch_shapes=[pltpu.VMEM((1, LAT), jnp.float32),
                        pltpu.VMEM((F_L, H), jnp.float32),
                        pltpu.VMEM((BATCH, H), jnp.float32),
                        pltpu.VMEM((1, H), jnp.float32)],
    )(hl, sl, Wl, hp, sp, Wp, lb3, pb3, to, Wle, ble, *tlws, *tpws,
      Wm, bm, Wv, bv, eps, Wlat, blat)


def kernel(ligand_x, pocket_x, type_output, eps, ligand_batch, pocket_batch,
           ll_edge_index, pp_edge_index, params):
    p = params
    r2 = lambda b: b.reshape(1, -1)

    def idx3(row):
        return row.astype(jnp.int32).reshape(_TILES, _NPASS, _WIN, _CH)

    lsrc = idx3(ll_edge_index[0])
    ldst = idx3(ll_edge_index[1])
    psrc = idx3(pp_edge_index[0])
    pdst = idx3(pp_edge_index[1])

    h_l, h_p = _embed_call(ligand_x, pocket_x,
                           p["Wle"], r2(p["ble"]), p["Wpe"], r2(p["bpe"]))

    for i in range(NLAYERS - 1):
        s_l, s_p = _sc_seg(h_l, h_p, lsrc, ldst, psrc, pdst)
        h_l, h_p = _update_call(h_l, s_l, p["Wg_l"][i], h_p, s_p, p["Wg_p"][i])
    s_l, s_p = _sc_seg(h_l, h_p, lsrc, ldst, psrc, pdst)

    lb3 = ligand_batch.astype(jnp.int32).reshape(_GRID, 1, _RB)
    pb3 = pocket_batch.astype(jnp.int32).reshape(_GRID, 1, _RB)

    def head_ws(pre):
        return tuple(x for j in range(3)
                     for x in (p[pre + "_W%d" % j], r2(p[pre + "_b%d" % j])))

    tll, tll_loss, dll, tlp, tlp_loss, dlp, vae = _final_call(
        h_l, s_l, p["Wg_l"][NLAYERS - 1], h_p, s_p, p["Wg_p"][NLAYERS - 1],
        lb3, pb3, type_output, p["Wle"], r2(p["ble"]),
        head_ws("ntll") + head_ws("ndll"), head_ws("ntlp") + head_ws("ndlp"),
        p["Wm"], r2(p["bm"]), p["Wv"], r2(p["bv"]), eps,
        p["Wlat"], r2(p["blat"]))
    vae_loss = vae.reshape(1)

    return (tll, tlp, tll_loss, tlp_loss, dll, dlp, vae_loss)
